# Initial kernel scaffold; baseline (speedup 1.0000x reference)
import jax
import jax.numpy as jnp
from jax.experimental import pallas as pl


def kernel(x, edge_index, edge_attr, We, be, W1, as1, ad1, b1, W2, as2, ad2, b2, W3, as3, ad3, b3, W4, as4, ad4, b4):
    raise NotImplementedError("write your pallas kernel here")


# speedup vs baseline: 66.0039x; 66.0039x over previous
import functools

import jax
import jax.numpy as jnp
import numpy as np
from jax import lax
from jax.experimental import pallas as pl
from jax.experimental.pallas import tpu as pltpu
from jax.experimental.pallas import tpu_sc as plsc

N = 10000
E = 320000
D = 128
HS = 144
NC = 2
NS = 16
NW = NC * NS
EPT = E // NW
K = 80
NCHUNK = EPT // K
NP = 10240
RPT = NP // NS
ZR = 128
NZ = RPT // ZR

f32 = jnp.float32
i32 = jnp.int32

BN = 1000
GRID = N // BN


def _sc_pre_body(ea_hbm, dst_hbm, s_out, eav, dstv, z32, s_sh):
    cid = lax.axis_index("c")
    sid = lax.axis_index("s")
    wid = sid * NC + cid

    def zfill(i, c):
        zv = jnp.zeros((16,), f32)
        z32[i, pl.ds(0, 16)] = zv
        z32[i, pl.ds(16, 16)] = zv
        return c
    lax.fori_loop(0, ZR, zfill, 0)

    r0 = sid * RPT

    def zcopy(i, c):
        pltpu.sync_copy(z32, s_sh.at[pl.ds(r0 + i * ZR, ZR)])
        return c
    lax.fori_loop(0, NZ, zcopy, 0)
    plsc.subcore_barrier()

    ebase = wid * EPT

    def chunk(ci, c):
        b = ebase + ci * K
        pltpu.sync_copy(dst_hbm.at[pl.ds(b, K)], dstv)
        pltpu.sync_copy(ea_hbm.at[pl.ds(b, K)], eav)
        pltpu.sync_copy(eav, s_sh.at[dstv], add=True)
        return c
    lax.fori_loop(0, NCHUNK, chunk, 0)
    plsc.subcore_barrier()

    def cout(i, c):
        pltpu.sync_copy(s_sh.at[pl.ds(r0 + i * ZR, ZR)],
                        s_out.at[cid, pl.ds(r0 + i * ZR, ZR)])
        return c
    lax.fori_loop(0, NZ, cout, 0)


_sc_pre = functools.partial(
    pl.kernel,
    mesh=plsc.VectorSubcoreMesh(core_axis_name="c", subcore_axis_name="s"),
    out_type=jax.ShapeDtypeStruct((NC, NP, 32), f32),
    scratch_types=[
        pltpu.VMEM((K, 32), f32),
        pltpu.VMEM((K,), i32),
        pltpu.VMEM((ZR, 32), f32),
        pltpu.VMEM_SHARED((NP, 32), f32),
    ],
    compiler_params=pltpu.CompilerParams(use_tc_tiling_on_sc=False),
)(_sc_pre_body)


def _sc_gat_body(src_hbm, dst_hbm, hs_hbm, aD_hbm, acc_out,
                 srcv, dstv, hrows, rowD, zbuf, acc_sh, s1, s2):
    cid = lax.axis_index("c")
    sid = lax.axis_index("s")
    wid = sid * NC + cid

    def zfill(i, c):
        zv = jnp.zeros((16,), f32)
        for j in range(9):
            zbuf[i, pl.ds(16 * j, 16)] = zv
        return c
    lax.fori_loop(0, ZR, zfill, 0)

    r0 = sid * RPT

    def zcopy(i, c):
        pltpu.sync_copy(zbuf, acc_sh.at[pl.ds(r0 + i * ZR, ZR)])
        return c
    lax.fori_loop(0, NZ, zcopy, 0)
    plsc.subcore_barrier()

    ebase = wid * EPT

    def chunk(ci, c):
        b = ebase + ci * K
        pltpu.sync_copy(src_hbm.at[pl.ds(b, K)], srcv)
        pltpu.sync_copy(dst_hbm.at[pl.ds(b, K)], dstv)
        g1 = pltpu.async_copy(hs_hbm.at[srcv], hrows, s1)
        g2 = pltpu.async_copy(aD_hbm.at[dstv], rowD, s2)
        g1.wait()
        g2.wait()

        @plsc.parallel_loop(0, K, unroll=2)
        def ebody(k):
            t = hrows[k, pl.ds(D, 16)] + rowD[k, :]
            v = jnp.exp(jnp.maximum(t, 0.2 * t))
            hrows[k, pl.ds(D, 16)] = v
            for j in range(8):
                hrows[k, pl.ds(16 * j, 16)] = hrows[k, pl.ds(16 * j, 16)] * v

        pltpu.sync_copy(hrows, acc_sh.at[dstv], add=True)
        return c
    lax.fori_loop(0, NCHUNK, chunk, 0)
    plsc.subcore_barrier()

    def cout(i, c):
        pltpu.sync_copy(acc_sh.at[pl.ds(r0 + i * ZR, ZR)],
                        acc_out.at[cid, pl.ds(r0 + i * ZR, ZR)])
        return c
    lax.fori_loop(0, NZ, cout, 0)


_sc_gat = functools.partial(
    pl.kernel,
    mesh=plsc.VectorSubcoreMesh(core_axis_name="c", subcore_axis_name="s"),
    out_type=jax.ShapeDtypeStruct((NC, NP, HS), f32),
    scratch_types=[
        pltpu.VMEM((K,), i32),
        pltpu.VMEM((K,), i32),
        pltpu.VMEM((K, HS), f32),
        pltpu.VMEM((K, 16), f32),
        pltpu.VMEM((ZR, HS), f32),
        pltpu.VMEM_SHARED((NP, HS), f32),
        pltpu.SemaphoreType.DMA,
        pltpu.SemaphoreType.DMA,
    ],
    compiler_params=pltpu.CompilerParams(use_tc_tiling_on_sc=False),
)(_sc_gat_body)


def _tc_pre1_body(x_ref, s_ref, We_ref, be_ref, W_ref, As_ref, Ad_ref,
                  hs_ref, aD_ref):
    s = s_ref[0] + s_ref[1]
    cnt = s[:, 16:17]
    e2n = jnp.dot(s[:, 0:16] / jnp.maximum(cnt, 1.0), We_ref[...],
                  preferred_element_type=f32)
    e2n = e2n + be_ref[...] * jnp.minimum(cnt, 1.0)
    xn = x_ref[...] + e2n
    h = jnp.dot(xn, W_ref[...], preferred_element_type=f32)
    hs_ref[:, 0:D] = h
    hs_ref[:, D:HS] = jnp.dot(h, As_ref[...], preferred_element_type=f32)
    aD_ref[...] = jnp.dot(h, Ad_ref[...], preferred_element_type=f32)


_tc_pre1 = pl.pallas_call(
    _tc_pre1_body,
    grid=(GRID,),
    in_specs=[
        pl.BlockSpec((BN, D), lambda i: (i, 0)),
        pl.BlockSpec((NC, BN, 32), lambda i: (0, i, 0)),
        pl.BlockSpec((16, D), lambda i: (0, 0)),
        pl.BlockSpec((1, D), lambda i: (0, 0)),
        pl.BlockSpec((D, D), lambda i: (0, 0)),
        pl.BlockSpec((D, 16), lambda i: (0, 0)),
        pl.BlockSpec((D, 16), lambda i: (0, 0)),
    ],
    out_specs=(pl.BlockSpec((BN, HS), lambda i: (i, 0)),
               pl.BlockSpec((BN, 16), lambda i: (i, 0))),
    out_shape=(jax.ShapeDtypeStruct((N, HS), f32),
               jax.ShapeDtypeStruct((N, 16), f32)),
)


def _tc_step_body(hs_ref, aD_ref, acc_ref, b_ref, W_ref,
                  As_ref, Ad_ref, hs2_ref, aD2_ref):
    h = hs_ref[:, 0:D]
    t = hs_ref[:, D:HS] + aD_ref[...]
    v = jnp.exp(jnp.maximum(t, 0.2 * t))
    vexp = jnp.concatenate([v] * 8, axis=1)
    num = acc_ref[0, :, 0:D] + acc_ref[1, :, 0:D] + h * vexp
    den = acc_ref[0, :, D:HS] + acc_ref[1, :, D:HS] + v
    dexp = jnp.concatenate([den] * 8, axis=1)
    xn = num / (dexp + 1e-16)
    xn = jnp.maximum(xn + b_ref[...], 0.0)
    h2 = jnp.dot(xn, W_ref[...], preferred_element_type=f32)
    hs2_ref[:, 0:D] = h2
    hs2_ref[:, D:HS] = jnp.dot(h2, As_ref[...], preferred_element_type=f32)
    aD2_ref[...] = jnp.dot(h2, Ad_ref[...], preferred_element_type=f32)


_tc_step = pl.pallas_call(
    _tc_step_body,
    grid=(GRID,),
    in_specs=[
        pl.BlockSpec((BN, HS), lambda i: (i, 0)),
        pl.BlockSpec((BN, 16), lambda i: (i, 0)),
        pl.BlockSpec((NC, BN, HS), lambda i: (0, i, 0)),
        pl.BlockSpec((1, D), lambda i: (0, 0)),
        pl.BlockSpec((D, D), lambda i: (0, 0)),
        pl.BlockSpec((D, 16), lambda i: (0, 0)),
        pl.BlockSpec((D, 16), lambda i: (0, 0)),
    ],
    out_specs=(pl.BlockSpec((BN, HS), lambda i: (i, 0)),
               pl.BlockSpec((BN, 16), lambda i: (i, 0))),
    out_shape=(jax.ShapeDtypeStruct((N, HS), f32),
               jax.ShapeDtypeStruct((N, 16), f32)),
)


def _tc_final_body(hs_ref, aD_ref, acc_ref, b_ref, o_ref):
    i = pl.program_id(0)
    h = hs_ref[:, 0:D]
    t = hs_ref[:, D:HS] + aD_ref[...]
    v = jnp.exp(jnp.maximum(t, 0.2 * t))
    v1 = v[:, 0:1]
    num = acc_ref[0, :, 0:D] + acc_ref[1, :, 0:D] + h * v1
    den = acc_ref[0, :, D:D + 1] + acc_ref[1, :, D:D + 1] + v1
    out = num / (den + 1e-16) + b_ref[...]

    @pl.when(i == 0)
    def _():
        o_ref[...] = jnp.zeros_like(o_ref)

    o_ref[...] += jnp.sum(out, axis=0, keepdims=True) * (1.0 / N)


_tc_final = pl.pallas_call(
    _tc_final_body,
    grid=(GRID,),
    in_specs=[
        pl.BlockSpec((BN, HS), lambda i: (i, 0)),
        pl.BlockSpec((BN, 16), lambda i: (i, 0)),
        pl.BlockSpec((NC, BN, HS), lambda i: (0, i, 0)),
        pl.BlockSpec((1, D), lambda i: (0, 0)),
    ],
    out_specs=pl.BlockSpec((1, D), lambda i: (0, 0)),
    out_shape=jax.ShapeDtypeStruct((1, D), f32),
    compiler_params=pltpu.CompilerParams(
        dimension_semantics=("arbitrary",)),
)


_R = np.arange(D)
_COLPERM = (_R % 8) * 16 + _R // 8


def _mk_alpha_rep(a, heads):
    A = jnp.zeros((D, 16), f32)
    if heads == 8:
        vals = a[_R % 8, _R // 8]
        A = A.at[_R, _R % 8].set(vals)
        A = A.at[_R, 8 + _R % 8].set(vals)
    else:
        A = jnp.tile(a[0][:, None], (1, 16))
    return A


def kernel(x, edge_index, edge_attr, We, be, W1, as1, ad1, b1,
           W2, as2, ad2, b2, W3, as3, ad3, b3, W4, as4, ad4, b4):
    src = edge_index[0]
    dst = edge_index[1]
    ea32 = jnp.concatenate(
        [edge_attr, jnp.ones((E, 1), f32), jnp.zeros((E, 15), f32)], axis=1)
    Ws = [W1[:, _COLPERM], W2[_COLPERM][:, _COLPERM],
          W3[_COLPERM][:, _COLPERM], W4[_COLPERM]]
    As = [_mk_alpha_rep(as1, 8), _mk_alpha_rep(as2, 8),
          _mk_alpha_rep(as3, 8), _mk_alpha_rep(as4, 1)]
    Ad = [_mk_alpha_rep(ad1, 8), _mk_alpha_rep(ad2, 8),
          _mk_alpha_rep(ad3, 8), _mk_alpha_rep(ad4, 1)]
    bs = [b1[_COLPERM].reshape(1, D), b2[_COLPERM].reshape(1, D),
          b3[_COLPERM].reshape(1, D), b4.reshape(1, D)]

    s32 = _sc_pre(ea32, dst)
    hs, aD = _tc_pre1(x, s32, We, be.reshape(1, D), Ws[0], As[0], Ad[0])
    for l in range(3):
        acc = _sc_gat(src, dst, hs, aD)
        hs, aD = _tc_step(hs, aD, acc, bs[l], Ws[l + 1], As[l + 1], Ad[l + 1])
    acc = _sc_gat(src, dst, hs, aD)
    out = _tc_final(hs, aD, acc, bs[3])
    return out.reshape(D)


# speedup vs baseline: 124.0404x; 1.8793x over previous
import functools

import jax
import jax.numpy as jnp
import numpy as np
from jax import lax
from jax.experimental import pallas as pl
from jax.experimental.pallas import tpu as pltpu
from jax.experimental.pallas import tpu_sc as plsc

N = 10000
E = 320000
D = 128
HS = 144
NC = 2
NS = 16
NW = NC * NS
EPT = E // NW
K = 80
NCHUNK = EPT // K
NP = 10240
RPT = NP // NS
ZR = 128
NZ = RPT // ZR
NP2 = 10112
RPT2 = NP2 // NS
ZTAIL = RPT2 - (RPT2 // K) * K

f32 = jnp.float32
i32 = jnp.int32

BN = 1000
GRID = N // BN


def _sc_pre_body(ea_hbm, dst_hbm, s_out, eav, dstall, z32, s_sh, *sems):
    cid = lax.axis_index("c")
    sid = lax.axis_index("s")
    wid = sid * NC + cid

    def zfill(i, c):
        zv = jnp.zeros((16,), f32)
        z32[i, pl.ds(0, 16)] = zv
        z32[i, pl.ds(16, 16)] = zv
        return c
    lax.fori_loop(0, ZR, zfill, 0)

    r0 = sid * RPT

    def zcopy(i, c):
        pltpu.sync_copy(z32, s_sh.at[pl.ds(r0 + i * ZR, ZR)])
        return c
    lax.fori_loop(0, NZ, zcopy, 0)
    plsc.subcore_barrier()

    ebase = wid * EPT
    pltpu.sync_copy(dst_hbm.at[pl.ds(ebase, EPT)], dstall)

    def issue(ci, b):
        pltpu.async_copy(ea_hbm.at[pl.ds(ebase + ci * K, K)],
                         eav.at[b], sems[b])

    def process(ci, b):
        pltpu.make_async_copy(ea_hbm.at[pl.ds(ebase, K)],
                              eav.at[b], sems[b]).wait()
        pltpu.sync_copy(eav.at[b],
                        s_sh.at[dstall.at[pl.ds(ci * K, K)]], add=True)

        @pl.when(ci + 2 < NCHUNK)
        def _():
            issue(ci + 2, b)

    issue(0, 0)
    issue(1, 1)

    def chunk2(c2, c):
        for b in range(2):
            process(2 * c2 + b, b)
        return c
    lax.fori_loop(0, NCHUNK // 2, chunk2, 0)
    process(NCHUNK - 1, (NCHUNK - 1) % 2)
    plsc.subcore_barrier()

    def cout(i, c):
        pltpu.sync_copy(s_sh.at[pl.ds(r0 + i * ZR, ZR)],
                        s_out.at[cid, pl.ds(r0 + i * ZR, ZR)])
        return c
    lax.fori_loop(0, NZ, cout, 0)


_sc_pre = functools.partial(
    pl.kernel,
    mesh=plsc.VectorSubcoreMesh(core_axis_name="c", subcore_axis_name="s"),
    out_type=jax.ShapeDtypeStruct((NC, NP, 32), f32),
    scratch_types=[
        pltpu.VMEM((2, K, 32), f32),
        pltpu.VMEM((EPT,), i32),
        pltpu.VMEM((ZR, 32), f32),
        pltpu.VMEM_SHARED((NP, 32), f32),
        pltpu.SemaphoreType.DMA,
        pltpu.SemaphoreType.DMA,
    ],
    compiler_params=pltpu.CompilerParams(use_tc_tiling_on_sc=False),
)(_sc_pre_body)


def _sc_gat_body(src_hbm, dst_hbm, hs_hbm, aD_hbm, acc_out,
                 idxS, idxD, hrows, rowD, acc_sh,
                 si0, si1, sj0, sj1, sj2, sh0, sh1, sd0, sd1, ss0, ss1):
    cid = lax.axis_index("c")
    sid = lax.axis_index("s")
    wid = sid * NC + cid
    sis = (si0, si1)
    sjs = (sj0, sj1, sj2)
    shs = (sh0, sh1)
    sds = (sd0, sd1)
    sss = (ss0, ss1)

    h0 = hrows.at[0]

    def zfill(i, c):
        zv = jnp.zeros((16,), f32)
        for j in range(9):
            h0[i, pl.ds(16 * j, 16)] = zv
        return c
    lax.fori_loop(0, K, zfill, 0)

    r0 = sid * RPT2

    def zcopy(i, c):
        pltpu.sync_copy(h0, acc_sh.at[pl.ds(r0 + i * K, K)])
        return c
    lax.fori_loop(0, RPT2 // K, zcopy, 0)
    pltpu.sync_copy(h0.at[pl.ds(0, ZTAIL)],
                    acc_sh.at[pl.ds(r0 + (RPT2 // K) * K, ZTAIL)])
    plsc.subcore_barrier()

    ebase = wid * EPT

    def idx_issue(ci, s2, s3):
        sl = pl.ds(ebase + ci * K, K)
        pltpu.async_copy(src_hbm.at[sl], idxS.at[s2], sis[s2])
        pltpu.async_copy(dst_hbm.at[sl], idxD.at[s3], sjs[s3])

    def idx_wait(s2, s3):
        pltpu.make_async_copy(src_hbm.at[pl.ds(0, K)], idxS.at[s2],
                              sis[s2]).wait()
        pltpu.make_async_copy(dst_hbm.at[pl.ds(0, K)], idxD.at[s3],
                              sjs[s3]).wait()

    def gat_issue(s2, s3):
        pltpu.async_copy(hs_hbm.at[idxS.at[s2]], hrows.at[s2], shs[s2])
        pltpu.async_copy(aD_hbm.at[idxD.at[s3]], rowD.at[s2], sds[s2])

    def gat_wait(s2, s3):
        pltpu.make_async_copy(hs_hbm.at[idxS.at[s2]], hrows.at[s2],
                              shs[s2]).wait()
        pltpu.make_async_copy(aD_hbm.at[idxD.at[s3]], rowD.at[s2],
                              sds[s2]).wait()

    def sc_wait(s2, s3):
        pltpu.make_async_copy(hrows.at[s2], acc_sh.at[idxD.at[s3]],
                              sss[s2]).wait()

    pltpu.sync_copy(src_hbm.at[pl.ds(ebase, K)], idxS.at[0])
    pltpu.sync_copy(dst_hbm.at[pl.ds(ebase, K)], idxD.at[0])
    gat_issue(0, 0)
    idx_issue(1, 1, 1)

    def step(ci, q, do_next, do_prev, do_pre2):
        b2, b3 = q % 2, q % 3
        n2, n3 = (q + 1) % 2, (q + 1) % 3
        if do_next:
            idx_wait(n2, n3)
        if do_prev:
            sc_wait(n2, (q + 2) % 3)
        if do_next:
            gat_issue(n2, n3)
        gat_wait(b2, b3)
        hb = hrows.at[b2]
        rb = rowD.at[b2]

        @plsc.parallel_loop(0, K, unroll=2)
        def ebody(k):
            t = hb[k, pl.ds(D, 16)] + rb[k, :]
            v = jnp.exp(jnp.maximum(t, 0.2 * t))
            hb[k, pl.ds(D, 16)] = v
            for j in range(8):
                hb[k, pl.ds(16 * j, 16)] = hb[k, pl.ds(16 * j, 16)] * v

        pltpu.async_copy(hb, acc_sh.at[idxD.at[b3]], sss[b2], add=True)
        if do_pre2:
            idx_issue(ci + 2, (q + 2) % 2, (q + 2) % 3)

    for q in range(6):
        step(q, q, True, q >= 1, True)

    def chunk6(c6, c):
        for q in range(6):
            step(6 * c6 + q, q, True, True, True)
        return c
    lax.fori_loop(1, NCHUNK // 6, chunk6, 0)
    for ci in range(NCHUNK - 5, NCHUNK):
        step(ci, ci, ci + 1 < NCHUNK, True, ci + 2 < NCHUNK)
    sc_wait((NCHUNK - 1) % 2, (NCHUNK - 1) % 3)
    plsc.subcore_barrier()

    def cout(i, c):
        pltpu.sync_copy(acc_sh.at[pl.ds(r0 + i * K, K)],
                        acc_out.at[cid, pl.ds(r0 + i * K, K)])
        return c
    lax.fori_loop(0, RPT2 // K, cout, 0)
    pltpu.sync_copy(acc_sh.at[pl.ds(r0 + (RPT2 // K) * K, ZTAIL)],
                    acc_out.at[cid, pl.ds(r0 + (RPT2 // K) * K, ZTAIL)])


_sc_gat = functools.partial(
    pl.kernel,
    mesh=plsc.VectorSubcoreMesh(core_axis_name="c", subcore_axis_name="s"),
    out_type=jax.ShapeDtypeStruct((NC, NP2, HS), f32),
    scratch_types=[
        pltpu.VMEM((2, K), i32),
        pltpu.VMEM((3, K), i32),
        pltpu.VMEM((2, K, HS), f32),
        pltpu.VMEM((2, K, 16), f32),
        pltpu.VMEM_SHARED((NP2, HS), f32),
        pltpu.SemaphoreType.DMA,
        pltpu.SemaphoreType.DMA,
        pltpu.SemaphoreType.DMA,
        pltpu.SemaphoreType.DMA,
        pltpu.SemaphoreType.DMA,
        pltpu.SemaphoreType.DMA,
        pltpu.SemaphoreType.DMA,
        pltpu.SemaphoreType.DMA,
        pltpu.SemaphoreType.DMA,
        pltpu.SemaphoreType.DMA,
        pltpu.SemaphoreType.DMA,
    ],
    compiler_params=pltpu.CompilerParams(use_tc_tiling_on_sc=False),
)(_sc_gat_body)


def _tc_pre1_body(x_ref, s_ref, We_ref, be_ref, W_ref, As_ref, Ad_ref,
                  hs_ref, aD_ref):
    s = s_ref[0] + s_ref[1]
    cnt = s[:, 16:17]
    e2n = jnp.dot(s[:, 0:16] / jnp.maximum(cnt, 1.0), We_ref[...],
                  preferred_element_type=f32)
    e2n = e2n + be_ref[...] * jnp.minimum(cnt, 1.0)
    xn = x_ref[...] + e2n
    h = jnp.dot(xn, W_ref[...], preferred_element_type=f32)
    hs_ref[:, 0:D] = h
    hs_ref[:, D:HS] = jnp.dot(h, As_ref[...], preferred_element_type=f32)
    aD_ref[...] = jnp.dot(h, Ad_ref[...], preferred_element_type=f32)


_tc_pre1 = pl.pallas_call(
    _tc_pre1_body,
    grid=(GRID,),
    in_specs=[
        pl.BlockSpec((BN, D), lambda i: (i, 0)),
        pl.BlockSpec((NC, BN, 32), lambda i: (0, i, 0)),
        pl.BlockSpec((16, D), lambda i: (0, 0)),
        pl.BlockSpec((1, D), lambda i: (0, 0)),
        pl.BlockSpec((D, D), lambda i: (0, 0)),
        pl.BlockSpec((D, 16), lambda i: (0, 0)),
        pl.BlockSpec((D, 16), lambda i: (0, 0)),
    ],
    out_specs=(pl.BlockSpec((BN, HS), lambda i: (i, 0)),
               pl.BlockSpec((BN, 16), lambda i: (i, 0))),
    out_shape=(jax.ShapeDtypeStruct((N, HS), f32),
               jax.ShapeDtypeStruct((N, 16), f32)),
)


def _tc_step_body(hs_ref, aD_ref, acc_ref, b_ref, W_ref,
                  As_ref, Ad_ref, hs2_ref, aD2_ref):
    h = hs_ref[:, 0:D]
    t = hs_ref[:, D:HS] + aD_ref[...]
    v = jnp.exp(jnp.maximum(t, 0.2 * t))
    vexp = jnp.concatenate([v] * 8, axis=1)
    num = acc_ref[0, :, 0:D] + acc_ref[1, :, 0:D] + h * vexp
    den = acc_ref[0, :, D:HS] + acc_ref[1, :, D:HS] + v
    dexp = jnp.concatenate([den] * 8, axis=1)
    xn = num / (dexp + 1e-16)
    xn = jnp.maximum(xn + b_ref[...], 0.0)
    h2 = jnp.dot(xn, W_ref[...], preferred_element_type=f32)
    hs2_ref[:, 0:D] = h2
    hs2_ref[:, D:HS] = jnp.dot(h2, As_ref[...], preferred_element_type=f32)
    aD2_ref[...] = jnp.dot(h2, Ad_ref[...], preferred_element_type=f32)


_tc_step = pl.pallas_call(
    _tc_step_body,
    grid=(GRID,),
    in_specs=[
        pl.BlockSpec((BN, HS), lambda i: (i, 0)),
        pl.BlockSpec((BN, 16), lambda i: (i, 0)),
        pl.BlockSpec((NC, BN, HS), lambda i: (0, i, 0)),
        pl.BlockSpec((1, D), lambda i: (0, 0)),
        pl.BlockSpec((D, D), lambda i: (0, 0)),
        pl.BlockSpec((D, 16), lambda i: (0, 0)),
        pl.BlockSpec((D, 16), lambda i: (0, 0)),
    ],
    out_specs=(pl.BlockSpec((BN, HS), lambda i: (i, 0)),
               pl.BlockSpec((BN, 16), lambda i: (i, 0))),
    out_shape=(jax.ShapeDtypeStruct((N, HS), f32),
               jax.ShapeDtypeStruct((N, 16), f32)),
)


def _tc_final_body(hs_ref, aD_ref, acc_ref, b_ref, o_ref):
    i = pl.program_id(0)
    h = hs_ref[:, 0:D]
    t = hs_ref[:, D:HS] + aD_ref[...]
    v = jnp.exp(jnp.maximum(t, 0.2 * t))
    v1 = v[:, 0:1]
    num = acc_ref[0, :, 0:D] + acc_ref[1, :, 0:D] + h * v1
    den = acc_ref[0, :, D:D + 1] + acc_ref[1, :, D:D + 1] + v1
    out = num / (den + 1e-16) + b_ref[...]

    @pl.when(i == 0)
    def _():
        o_ref[...] = jnp.zeros_like(o_ref)

    o_ref[...] += jnp.sum(out, axis=0, keepdims=True) * (1.0 / N)


_tc_final = pl.pallas_call(
    _tc_final_body,
    grid=(GRID,),
    in_specs=[
        pl.BlockSpec((BN, HS), lambda i: (i, 0)),
        pl.BlockSpec((BN, 16), lambda i: (i, 0)),
        pl.BlockSpec((NC, BN, HS), lambda i: (0, i, 0)),
        pl.BlockSpec((1, D), lambda i: (0, 0)),
    ],
    out_specs=pl.BlockSpec((1, D), lambda i: (0, 0)),
    out_shape=jax.ShapeDtypeStruct((1, D), f32),
    compiler_params=pltpu.CompilerParams(
        dimension_semantics=("arbitrary",)),
)


_R = np.arange(D)
_COLPERM = (_R % 8) * 16 + _R // 8


def _mk_alpha_rep(a, heads):
    A = jnp.zeros((D, 16), f32)
    if heads == 8:
        vals = a[_R % 8, _R // 8]
        A = A.at[_R, _R % 8].set(vals)
        A = A.at[_R, 8 + _R % 8].set(vals)
    else:
        A = jnp.tile(a[0][:, None], (1, 16))
    return A


def kernel(x, edge_index, edge_attr, We, be, W1, as1, ad1, b1,
           W2, as2, ad2, b2, W3, as3, ad3, b3, W4, as4, ad4, b4):
    src = edge_index[0]
    dst = edge_index[1]
    ea32 = jnp.concatenate(
        [edge_attr, jnp.ones((E, 1), f32), jnp.zeros((E, 15), f32)], axis=1)
    Ws = [W1[:, _COLPERM], W2[_COLPERM][:, _COLPERM],
          W3[_COLPERM][:, _COLPERM], W4[_COLPERM]]
    As = [_mk_alpha_rep(as1, 8), _mk_alpha_rep(as2, 8),
          _mk_alpha_rep(as3, 8), _mk_alpha_rep(as4, 1)]
    Ad = [_mk_alpha_rep(ad1, 8), _mk_alpha_rep(ad2, 8),
          _mk_alpha_rep(ad3, 8), _mk_alpha_rep(ad4, 1)]
    bs = [b1[_COLPERM].reshape(1, D), b2[_COLPERM].reshape(1, D),
          b3[_COLPERM].reshape(1, D), b4.reshape(1, D)]

    s32 = _sc_pre(ea32, dst)
    hs, aD = _tc_pre1(x, s32, We, be.reshape(1, D), Ws[0], As[0], Ad[0])
    for l in range(3):
        acc = _sc_gat(src, dst, hs, aD)
        hs, aD = _tc_step(hs, aD, acc, bs[l], Ws[l + 1], As[l + 1], Ad[l + 1])
    acc = _sc_gat(src, dst, hs, aD)
    out = _tc_final(hs, aD, acc, bs[3])
    return out.reshape(D)
